# Initial kernel scaffold; baseline (speedup 1.0000x reference)
import jax
import jax.numpy as jnp
from jax.experimental import pallas as pl


def kernel(event_ids, event_embeddings_weight):
    raise NotImplementedError("write your pallas kernel here")


# speedup vs baseline: 1.7987x; 1.7987x over previous
import functools

import jax
import jax.numpy as jnp
from jax import lax
from jax.experimental import pallas as pl
from jax.experimental.pallas import tpu as pltpu
from jax.experimental.pallas import tpu_sc as plsc

_B = 16384 * 50
_D = 64
_NC = 2
_NS = 16
_NW = _NC * _NS
_BPW = _B // _NW
_C = 512
_NCHUNK = _BPW // _C


def _emb_body(idx_hbm, table_hbm, out_hbm, idx_v, rows_v, sem):
    wid = lax.axis_index("s") * _NC + lax.axis_index("c")
    base = wid * _BPW

    def body(i, carry):
        off = base + i * _C
        pltpu.sync_copy(idx_hbm.at[pl.ds(off, _C)], idx_v)
        pltpu.async_copy(table_hbm.at[idx_v], rows_v, sem).wait()
        pltpu.sync_copy(rows_v, out_hbm.at[pl.ds(off, _C)])
        return carry

    lax.fori_loop(0, _NCHUNK, body, 0)


@functools.partial(jax.jit, static_argnums=())
def _emb(idx, table):
    mesh = plsc.VectorSubcoreMesh(core_axis_name="c", subcore_axis_name="s")
    f = functools.partial(
        pl.kernel,
        mesh=mesh,
        out_type=jax.ShapeDtypeStruct((_B, _D), jnp.float32),
        scratch_types=[
            pltpu.VMEM((_C,), jnp.int32),
            pltpu.VMEM((_C, _D), jnp.float32),
            pltpu.SemaphoreType.DMA,
        ],
        compiler_params=pltpu.CompilerParams(use_tc_tiling_on_sc=False),
    )(_emb_body)
    return f(idx, table)


def kernel(event_ids, event_embeddings_weight):
    idx = event_ids.reshape(-1).astype(jnp.int32)
    out = _emb(idx, event_embeddings_weight)
    return out.reshape(event_ids.shape + (_D,))


# speedup vs baseline: 1.8756x; 1.0427x over previous
import functools

import jax
import jax.numpy as jnp
from jax import lax
from jax.experimental import pallas as pl
from jax.experimental.pallas import tpu as pltpu
from jax.experimental.pallas import tpu_sc as plsc

_B = 16384 * 50
_D = 64
_NC = 2
_NS = 16
_NW = _NC * _NS
_BPW = _B // _NW
_C = 256
_NCHUNK = _BPW // _C
_NBUF = 4


def _emb_body(idx_hbm, table_hbm, out_hbm, idx_v, *scratch):
    rows = list(scratch[:_NBUF])
    gsems = list(scratch[_NBUF:2 * _NBUF])
    osems = list(scratch[2 * _NBUF:3 * _NBUF])

    wid = lax.axis_index("s") * _NC + lax.axis_index("c")
    base = wid * _BPW
    pltpu.sync_copy(idx_hbm.at[wid], idx_v)

    def issue_gather(i, b):
        pltpu.make_async_copy(
            table_hbm.at[idx_v.at[i]], rows[b], gsems[b]).start()

    def wait_gather(b):
        pltpu.make_async_copy(
            table_hbm.at[idx_v.at[0]], rows[b], gsems[b]).wait()

    def issue_store(i, b):
        pltpu.make_async_copy(
            rows[b], out_hbm.at[pl.ds(base + i * _C, _C)], osems[b]).start()

    def wait_store(b):
        pltpu.make_async_copy(
            rows[b], out_hbm.at[pl.ds(base, _C)], osems[b]).wait()

    for g in range(_NBUF - 1):
        issue_gather(g, g)
    issue_gather(_NBUF - 1, _NBUF - 1)
    wait_gather(0)
    issue_store(0, 0)

    n_main = _NCHUNK - _NBUF
    def outer(k, carry):
        i0 = 1 + k * _NBUF
        for t in range(_NBUF):
            i = i0 + t
            b = (1 + t) % _NBUF
            bp = t % _NBUF
            wait_store(bp)
            issue_gather(i + _NBUF - 1, bp)
            wait_gather(b)
            issue_store(i, b)
        return carry

    lax.fori_loop(0, n_main // _NBUF, outer, 0)

    for i in range(_NCHUNK - _NBUF + 1, _NCHUNK):
        b = i % _NBUF
        wait_gather(b)
        issue_store(i, b)
    for b in range(_NBUF):
        wait_store(b)


@jax.jit
def _emb(idx, table):
    mesh = plsc.VectorSubcoreMesh(core_axis_name="c", subcore_axis_name="s")
    f = functools.partial(
        pl.kernel,
        mesh=mesh,
        out_type=jax.ShapeDtypeStruct((_B, _D), jnp.float32),
        scratch_types=(
            [pltpu.VMEM((_NCHUNK, _C), jnp.int32)]
            + [pltpu.VMEM((_C, _D), jnp.float32) for _ in range(_NBUF)]
            + [pltpu.SemaphoreType.DMA for _ in range(2 * _NBUF)]
        ),
        compiler_params=pltpu.CompilerParams(use_tc_tiling_on_sc=False),
    )(_emb_body)
    return f(idx, table)


def kernel(event_ids, event_embeddings_weight):
    idx = event_ids.reshape(-1).astype(jnp.int32).reshape(_NW, _NCHUNK, _C)
    out = _emb(idx, event_embeddings_weight)
    return out.reshape(event_ids.shape + (_D,))
